# Initial kernel scaffold; baseline (speedup 1.0000x reference)
import jax
import jax.numpy as jnp
from jax.experimental import pallas as pl


def kernel(node_features, parent1, parent2, W1a, b1a, g1i, be1i, W1b, b1b, g1o, be1o, W2a, b2a, g2i, be2i, W2b, b2b, g2o, be2o, Wp, bp):
    raise NotImplementedError("write your pallas kernel here")


# speedup vs baseline: 3.4247x; 3.4247x over previous
import functools

import jax
import jax.numpy as jnp
from jax import lax
from jax.experimental import pallas as pl
from jax.experimental.pallas import tpu as pltpu
from jax.experimental.pallas import tpu_sc as plsc

N0, N1, N2 = 100000, 25000, 6250
D = 128
HALF = 64
K = 200
ZK = 200
NS = 16
EPS = 1e-5


def _make_scatter(n_in, n_out):
    n_chunks = n_in // K
    assert n_in % K == 0
    nf = n_out // K
    rem = n_out - nf * K
    nzf = n_out // ZK
    zrem = n_out - nzf * ZK
    nmax = -(-n_chunks // NS)
    nmaxz = -(-nzf // NS)
    nmaxw = -(-nf // NS)
    mesh = plsc.VectorSubcoreMesh(core_axis_name="c", subcore_axis_name="s")

    @functools.partial(
        pl.kernel,
        mesh=mesh,
        out_type=jax.ShapeDtypeStruct((n_out, D), jnp.float32),
        scratch_types=[
            pltpu.VMEM((K, HALF), jnp.float32),
            pltpu.VMEM((K,), jnp.int32),
            pltpu.VMEM_SHARED((n_out, HALF), jnp.float32),
        ],
        compiler_params=pltpu.CompilerParams(use_tc_tiling_on_sc=False),
    )
    def scat(feats, parent, out, buf, idxb, acc):
        c = lax.axis_index("c")
        s = lax.axis_index("s")
        col0 = c * HALF

        zv = jnp.zeros((16,), jnp.float32)

        def zstore(i, _):
            for j4 in range(HALF // 16):
                buf[i, pl.ds(j4 * 16, 16)] = zv
            return 0

        lax.fori_loop(0, ZK, zstore, 0)
        for j in range(nmaxz):
            chunk = s + j * NS

            @pl.when(chunk < nzf)
            def _():
                pltpu.sync_copy(buf.at[pl.ds(0, ZK)], acc.at[pl.ds(chunk * ZK, ZK)])
        if zrem:
            @pl.when(s == NS - 1)
            def _():
                pltpu.sync_copy(buf.at[pl.ds(0, zrem)], acc.at[pl.ds(nzf * ZK, zrem)])
        plsc.subcore_barrier()

        for j in range(nmax):
            chunk = s + j * NS

            @pl.when(chunk < n_chunks)
            def _():
                r0 = chunk * K
                pltpu.sync_copy(parent.at[pl.ds(r0, K)], idxb)
                pltpu.sync_copy(feats.at[pl.ds(r0, K), pl.ds(col0, HALF)], buf)
                pltpu.sync_copy(buf, acc.at[idxb], add=True)
        plsc.subcore_barrier()

        for j in range(nmaxw):
            chunk = s + j * NS

            @pl.when(chunk < nf)
            def _():
                r0 = chunk * K
                pltpu.sync_copy(acc.at[pl.ds(r0, K)], buf)
                pltpu.sync_copy(buf, out.at[pl.ds(r0, K), pl.ds(col0, HALF)])
        if rem:
            @pl.when(s == NS - 1)
            def _():
                pltpu.sync_copy(acc.at[pl.ds(nf * K, rem)], buf.at[pl.ds(0, rem)])
                pltpu.sync_copy(buf.at[pl.ds(0, rem)],
                                out.at[pl.ds(nf * K, rem), pl.ds(col0, HALF)])

    return scat


_scatter1 = _make_scatter(N0, N1)
_scatter2 = _make_scatter(N1, N2)


def _dense_body(x_ref, wa_ref, ba_ref, gi_ref, bi_ref, wb_ref, bb_ref,
                go_ref, bo_ref, *rest):
    has_head = len(rest) == 3
    x = x_ref[...]
    y = jnp.dot(x, wa_ref[...], preferred_element_type=jnp.float32) + ba_ref[...]
    m = jnp.mean(y, axis=0, keepdims=True)
    v = jnp.mean((y - m) * (y - m), axis=0, keepdims=True)
    h = jnp.maximum((y - m) * lax.rsqrt(v + EPS) * gi_ref[...] + bi_ref[...], 0.0)
    z = jnp.dot(h, wb_ref[...], preferred_element_type=jnp.float32) + bb_ref[...]
    m2 = jnp.mean(z, axis=0, keepdims=True)
    v2 = jnp.mean((z - m2) * (z - m2), axis=0, keepdims=True)
    h2 = jnp.maximum((z - m2) * lax.rsqrt(v2 + EPS) * go_ref[...] + bo_ref[...], 0.0)
    if has_head:
        wp_ref, bp_ref, out_ref = rest
        out_ref[...] = (jnp.dot(h2, wp_ref[...], preferred_element_type=jnp.float32)
                        + bp_ref[...])
    else:
        out_ref, = rest
        out_ref[...] = h2


def _dense(x, wa, ba, gi, bi, wb, bb, go, bo):
    return pl.pallas_call(
        _dense_body,
        out_shape=jax.ShapeDtypeStruct(x.shape, jnp.float32),
    )(x, wa, ba, gi, bi, wb, bb, go, bo)


def _dense_head(x, wa, ba, gi, bi, wb, bb, go, bo, wp, bp):
    return pl.pallas_call(
        _dense_body,
        out_shape=jax.ShapeDtypeStruct((x.shape[0], wp.shape[1]), jnp.float32),
    )(x, wa, ba, gi, bi, wb, bb, go, bo, wp, bp)


def kernel(node_features, parent1, parent2,
           W1a, b1a, g1i, be1i, W1b, b1b, g1o, be1o,
           W2a, b2a, g2i, be2i, W2b, b2b, g2o, be2o,
           Wp, bp):
    p1 = parent1.astype(jnp.int32)
    p2 = parent2.astype(jnp.int32)
    r = lambda a: a.reshape(1, -1)
    pooled1 = _scatter1(node_features, p1)
    h1 = _dense(pooled1, W1a, r(b1a), r(g1i), r(be1i), W1b, r(b1b), r(g1o), r(be1o))
    pooled2 = _scatter2(h1, p2)
    return _dense_head(pooled2, W2a, r(b2a), r(g2i), r(be2i), W2b, r(b2b),
                       r(g2o), r(be2o), Wp, r(bp))


# speedup vs baseline: 4.9134x; 1.4347x over previous
import functools

import jax
import jax.numpy as jnp
from jax import lax
from jax.experimental import pallas as pl
from jax.experimental.pallas import tpu as pltpu
from jax.experimental.pallas import tpu_sc as plsc

N0, N1, N2 = 100000, 25000, 6250
D = 128
HALF = 64
K = 200
ZK = 200
NS = 16
EPS = 1e-5


def _make_scatter(n_in, n_out):
    n_chunks = n_in // K
    assert n_in % K == 0
    nf = n_out // K
    rem = n_out - nf * K
    nzf = n_out // ZK
    zrem = n_out - nzf * ZK
    nmax = -(-n_chunks // NS)
    nmaxz = -(-nzf // NS)
    nmaxw = -(-nf // NS)
    mesh = plsc.VectorSubcoreMesh(core_axis_name="c", subcore_axis_name="s")

    @functools.partial(
        pl.kernel,
        mesh=mesh,
        out_type=jax.ShapeDtypeStruct((n_out, D), jnp.float32),
        scratch_types=[
            pltpu.VMEM((K, HALF), jnp.float32),
            pltpu.VMEM((K, HALF), jnp.float32),
            pltpu.VMEM((K,), jnp.int32),
            pltpu.VMEM((K,), jnp.int32),
            pltpu.VMEM_SHARED((n_out, HALF), jnp.float32),
            pltpu.SemaphoreType.DMA,
            pltpu.SemaphoreType.DMA,
            pltpu.SemaphoreType.DMA,
        ],
        compiler_params=pltpu.CompilerParams(use_tc_tiling_on_sc=False),
    )
    def scat(feats, parent, out, buf0, buf1, idx0, idx1, acc,
             sem0, sem1, semz):
        c = lax.axis_index("c")
        s = lax.axis_index("s")
        col0 = c * HALF
        bufs, idxs, sems = (buf0, buf1), (idx0, idx1), (sem0, sem1)

        def loads(j, b):
            chunk = s + j * NS

            @pl.when(chunk < n_chunks)
            def _():
                r0 = chunk * K
                pltpu.make_async_copy(
                    parent.at[pl.ds(r0, K)], idxs[b], sems[b]).start()
                pltpu.make_async_copy(
                    feats.at[pl.ds(r0, K), pl.ds(col0, HALF)], bufs[b],
                    sems[b]).start()

        def wait_scatter(j, b):
            chunk = s + j * NS

            @pl.when(chunk < n_chunks)
            def _():
                r0 = chunk * K
                pltpu.make_async_copy(
                    parent.at[pl.ds(r0, K)], idxs[b], sems[b]).wait()
                pltpu.make_async_copy(
                    feats.at[pl.ds(r0, K), pl.ds(col0, HALF)], bufs[b],
                    sems[b]).wait()
                pltpu.sync_copy(bufs[b], acc.at[idxs[b]], add=True)

        zsrc = buf0.at[pl.ds(0, ZK)]
        zv = jnp.zeros((16,), jnp.float32)

        def zstore(i, _):
            for j4 in range(HALF // 16):
                buf0[i, pl.ds(j4 * 16, 16)] = zv
            return 0

        lax.fori_loop(0, ZK, zstore, 0)

        def zfire(j, _):
            chunk = s + j * NS

            @pl.when(chunk < nzf)
            def _():
                pltpu.make_async_copy(
                    zsrc, acc.at[pl.ds(chunk * ZK, ZK)], semz).start()
            return 0

        def zdrain(j, _):
            chunk = s + j * NS

            @pl.when(chunk < nzf)
            def _():
                pltpu.make_async_copy(
                    zsrc, acc.at[pl.ds(chunk * ZK, ZK)], semz).wait()
            return 0

        lax.fori_loop(0, nmaxz, zfire, 0)
        if zrem:
            @pl.when(s == NS - 1)
            def _():
                pltpu.make_async_copy(
                    buf0.at[pl.ds(0, zrem)], acc.at[pl.ds(nzf * ZK, zrem)],
                    semz).start()
        lax.fori_loop(0, nmaxz, zdrain, 0)
        if zrem:
            @pl.when(s == NS - 1)
            def _():
                pltpu.make_async_copy(
                    buf0.at[pl.ds(0, zrem)], acc.at[pl.ds(nzf * ZK, zrem)],
                    semz).wait()

        loads(0, 0)
        loads(1, 1)
        plsc.subcore_barrier()

        def pair(i, _):
            wait_scatter(2 * i, 0)
            loads(2 * i + 2, 0)
            wait_scatter(2 * i + 1, 1)
            loads(2 * i + 3, 1)
            return 0

        lax.fori_loop(0, -(-nmax // 2), pair, 0)
        plsc.subcore_barrier()

        def wdst(chunk):
            return out.at[pl.ds(chunk * K, K), pl.ds(col0, HALF)]

        def wout(j, b):
            chunk = s + j * NS

            @pl.when(chunk < nf)
            def _():
                @pl.when(j >= 2)
                def _():
                    pltpu.make_async_copy(bufs[b], wdst(chunk - 2 * NS),
                                          sems[b]).wait()
                pltpu.sync_copy(acc.at[pl.ds(chunk * K, K)], bufs[b])
                pltpu.make_async_copy(bufs[b], wdst(chunk), sems[b]).start()

        def wpair(i, _):
            wout(2 * i, 0)
            wout(2 * i + 1, 1)
            return 0

        lax.fori_loop(0, -(-nmaxw // 2), wpair, 0)
        jl = (nf - 1 - s) // NS
        for b in (0, 1):
            jb = jl - ((jl - b) & 1)

            @pl.when(jb >= 0)
            def _():
                pltpu.make_async_copy(bufs[b], wdst(s + jb * NS),
                                      sems[b]).wait()
        if rem:
            @pl.when(s == NS - 1)
            def _():
                pltpu.sync_copy(acc.at[pl.ds(nf * K, rem)],
                                buf0.at[pl.ds(0, rem)])
                pltpu.sync_copy(buf0.at[pl.ds(0, rem)],
                                out.at[pl.ds(nf * K, rem), pl.ds(col0, HALF)])

    return scat


_scatter1 = _make_scatter(N0, N1)
_scatter2 = _make_scatter(N1, N2)


def _dense_body(x_ref, wa_ref, ba_ref, gi_ref, bi_ref, wb_ref, bb_ref,
                go_ref, bo_ref, *rest):
    has_head = len(rest) == 3
    x = x_ref[...]
    y = jnp.dot(x, wa_ref[...], preferred_element_type=jnp.float32) + ba_ref[...]
    m = jnp.mean(y, axis=0, keepdims=True)
    v = jnp.mean((y - m) * (y - m), axis=0, keepdims=True)
    h = jnp.maximum((y - m) * lax.rsqrt(v + EPS) * gi_ref[...] + bi_ref[...], 0.0)
    z = jnp.dot(h, wb_ref[...], preferred_element_type=jnp.float32) + bb_ref[...]
    m2 = jnp.mean(z, axis=0, keepdims=True)
    v2 = jnp.mean((z - m2) * (z - m2), axis=0, keepdims=True)
    h2 = jnp.maximum((z - m2) * lax.rsqrt(v2 + EPS) * go_ref[...] + bo_ref[...], 0.0)
    if has_head:
        wp_ref, bp_ref, out_ref = rest
        out_ref[...] = (jnp.dot(h2, wp_ref[...], preferred_element_type=jnp.float32)
                        + bp_ref[...])
    else:
        out_ref, = rest
        out_ref[...] = h2


def _dense(x, wa, ba, gi, bi, wb, bb, go, bo):
    return pl.pallas_call(
        _dense_body,
        out_shape=jax.ShapeDtypeStruct(x.shape, jnp.float32),
    )(x, wa, ba, gi, bi, wb, bb, go, bo)


def _dense_head(x, wa, ba, gi, bi, wb, bb, go, bo, wp, bp):
    return pl.pallas_call(
        _dense_body,
        out_shape=jax.ShapeDtypeStruct((x.shape[0], wp.shape[1]), jnp.float32),
    )(x, wa, ba, gi, bi, wb, bb, go, bo, wp, bp)


def kernel(node_features, parent1, parent2,
           W1a, b1a, g1i, be1i, W1b, b1b, g1o, be1o,
           W2a, b2a, g2i, be2i, W2b, b2b, g2o, be2o,
           Wp, bp):
    p1 = parent1.astype(jnp.int32)
    p2 = parent2.astype(jnp.int32)
    r = lambda a: a.reshape(1, -1)
    pooled1 = _scatter1(node_features, p1)
    h1 = _dense(pooled1, W1a, r(b1a), r(g1i), r(be1i), W1b, r(b1b), r(g1o), r(be1o))
    pooled2 = _scatter2(h1, p2)
    return _dense_head(pooled2, W2a, r(b2a), r(g2i), r(be2i), W2b, r(b2b),
                       r(g2o), r(be2o), Wp, r(bp))
